# Initial kernel scaffold; baseline (speedup 1.0000x reference)
import jax
import jax.numpy as jnp
from jax.experimental import pallas as pl


def kernel(word_input, character_input, word_table, char_table, W1, W2, W3):
    raise NotImplementedError("write your pallas kernel here")


# speedup vs baseline: 10.5874x; 10.5874x over previous
import functools

import jax
import jax.numpy as jnp
from jax import lax
from jax.experimental import pallas as pl
from jax.experimental.pallas import tpu as pltpu
from jax.experimental.pallas import tpu_sc as plsc

_NC = 2
_NS = 16
_NW = _NC * _NS
_CH = 128


def _sc_word_gather(widx, table):
    n = widx.shape[0]
    per = n // _NW
    nch = per // _CH
    mesh = plsc.VectorSubcoreMesh(core_axis_name="c", subcore_axis_name="s")

    @functools.partial(
        pl.kernel,
        mesh=mesh,
        out_type=jax.ShapeDtypeStruct((n, 128), jnp.float32),
        scratch_types=[
            pltpu.VMEM((per,), jnp.int32),
            pltpu.VMEM((_CH, 128), jnp.float32),
            pltpu.SemaphoreType.DMA,
        ],
    )
    def k(idx_hbm, tab_hbm, out_hbm, idx_v, rows_v, sem):
        wid = lax.axis_index("s") * _NC + lax.axis_index("c")
        base = wid * per
        pltpu.sync_copy(idx_hbm.at[pl.ds(base, per)], idx_v)

        def body(j, carry):
            off = j * _CH
            pltpu.async_copy(tab_hbm.at[idx_v.at[pl.ds(off, _CH)]], rows_v, sem).wait()
            pltpu.sync_copy(rows_v, out_hbm.at[pl.ds(base + off, _CH)])
            return carry

        lax.fori_loop(0, nch, body, 0)

    return k(widx, table)


def _tdnn_body(ci_ref, we_ref, ct_ref, w1_ref, w2_ref, w3_ref, out_ref):
    r = ci_ref.shape[0]
    ci = ci_ref[...]
    cis = jnp.concatenate([ci[:, t : t + 1] for t in range(12)], axis=0)
    lanes = lax.broadcasted_iota(jnp.int32, (1, 128), 1)
    oh = (cis == lanes).astype(jnp.float32)
    e = jnp.dot(oh, ct_ref[...], preferred_element_type=jnp.float32)

    y1 = jnp.maximum(jnp.dot(e, w1_ref[...], preferred_element_type=jnp.float32), 0.0)
    m1 = y1[0:r]
    for t in range(1, 12):
        m1 = jnp.maximum(m1, y1[t * r : (t + 1) * r])

    x2 = jnp.concatenate([e[0 : 11 * r], e[r : 12 * r]], axis=1)
    y2 = jnp.maximum(jnp.dot(x2, w2_ref[...], preferred_element_type=jnp.float32), 0.0)
    m2 = y2[0:r]
    for t in range(1, 11):
        m2 = jnp.maximum(m2, y2[t * r : (t + 1) * r])

    x3 = jnp.concatenate(
        [e[0 : 10 * r], e[r : 11 * r], e[2 * r : 12 * r]], axis=1
    )
    y3 = jnp.maximum(jnp.dot(x3, w3_ref[...], preferred_element_type=jnp.float32), 0.0)
    m3 = y3[0:r]
    for t in range(1, 10):
        m3 = jnp.maximum(m3, y3[t * r : (t + 1) * r])

    out_ref[...] = jnp.concatenate([we_ref[...], m1, m2, m3], axis=1)


def _tdnn_call(ci, we, ct, w1f, w2f, w3f):
    n = ci.shape[0]
    r = 512
    return pl.pallas_call(
        _tdnn_body,
        grid=(n // r,),
        in_specs=[
            pl.BlockSpec((r, 12), lambda i: (i, 0)),
            pl.BlockSpec((r, 128), lambda i: (i, 0)),
            pl.BlockSpec((128, 16), lambda i: (0, 0)),
            pl.BlockSpec((16, 32), lambda i: (0, 0)),
            pl.BlockSpec((32, 32), lambda i: (0, 0)),
            pl.BlockSpec((48, 64), lambda i: (0, 0)),
        ],
        out_specs=pl.BlockSpec((r, 256), lambda i: (i, 0)),
        out_shape=jax.ShapeDtypeStruct((n, 256), jnp.float32),
    )(ci, we, ct, w1f, w2f, w3f)


def kernel(word_input, character_input, word_table, char_table, W1, W2, W3):
    b, s = word_input.shape
    n = b * s
    widx = word_input.reshape(n).astype(jnp.int32)
    we = _sc_word_gather(widx, word_table)

    ci = character_input.reshape(n, 12).astype(jnp.int32)
    w1f = W1.transpose(2, 1, 0).reshape(16, 32)
    w2f = W2.transpose(2, 1, 0).reshape(32, 32)
    w3f = W3.transpose(2, 1, 0).reshape(48, 64)
    out = _tdnn_call(ci, we, char_table, w1f, w2f, w3f)
    return out.reshape(b, s, 256)


# speedup vs baseline: 23.7039x; 2.2389x over previous
import functools

import jax
import jax.numpy as jnp
from jax import lax
from jax.experimental import pallas as pl
from jax.experimental.pallas import tpu as pltpu
from jax.experimental.pallas import tpu_sc as plsc

_NC = 2
_NS = 16
_NW = _NC * _NS
_CH = 128


def _sc_word_gather(widx, table):
    n = widx.shape[0]
    per = n // _NW
    nch = per // _CH
    mesh = plsc.VectorSubcoreMesh(core_axis_name="c", subcore_axis_name="s")

    @functools.partial(
        pl.kernel,
        mesh=mesh,
        out_type=jax.ShapeDtypeStruct((n, 128), jnp.float32),
        scratch_types=[
            pltpu.VMEM((per,), jnp.int32),
            pltpu.VMEM((_CH, 128), jnp.float32),
            pltpu.SemaphoreType.DMA,
        ],
    )
    def k(idx_hbm, tab_hbm, out_hbm, idx_v, rows_v, sem):
        wid = lax.axis_index("s") * _NC + lax.axis_index("c")
        base = wid * per
        pltpu.sync_copy(idx_hbm.at[pl.ds(base, per)], idx_v)

        def body(j, carry):
            off = j * _CH
            pltpu.async_copy(tab_hbm.at[idx_v.at[pl.ds(off, _CH)]], rows_v, sem).wait()
            pltpu.sync_copy(rows_v, out_hbm.at[pl.ds(base + off, _CH)])
            return carry

        lax.fori_loop(0, nch, body, 0)

    return k(widx, table)


def _tdnn_body(cit_ref, we_ref, ctt_ref, w1_ref, a2_ref, b2_ref, a3_ref,
               b3_ref, c3_ref, out_ref):
    r = cit_ref.shape[1]
    cit = cit_ref[...]
    subs = lax.broadcasted_iota(jnp.int32, (128, r), 0)
    ctt = ctt_ref[...]

    es = []
    for t in range(12):
        oh = (jnp.broadcast_to(cit[t : t + 1, :], (128, r)) == subs).astype(
            jnp.float32
        )
        es.append(jnp.dot(ctt, oh, preferred_element_type=jnp.float32))
    eT = jnp.concatenate(es, axis=1)

    y1 = jnp.maximum(
        jnp.dot(w1_ref[...], eT, preferred_element_type=jnp.float32), 0.0
    )
    m1 = y1[:, 0:r]
    for t in range(1, 12):
        m1 = jnp.maximum(m1, y1[:, t * r : (t + 1) * r])

    y2 = jnp.maximum(
        jnp.dot(a2_ref[...], eT[:, 0 : 11 * r], preferred_element_type=jnp.float32)
        + jnp.dot(b2_ref[...], eT[:, r : 12 * r], preferred_element_type=jnp.float32),
        0.0,
    )
    m2 = y2[:, 0:r]
    for t in range(1, 11):
        m2 = jnp.maximum(m2, y2[:, t * r : (t + 1) * r])

    y3 = jnp.maximum(
        jnp.dot(a3_ref[...], eT[:, 0 : 10 * r], preferred_element_type=jnp.float32)
        + jnp.dot(b3_ref[...], eT[:, r : 11 * r], preferred_element_type=jnp.float32)
        + jnp.dot(c3_ref[...], eT[:, 2 * r : 12 * r], preferred_element_type=jnp.float32),
        0.0,
    )
    m3 = y3[:, 0:r]
    for t in range(1, 10):
        m3 = jnp.maximum(m3, y3[:, t * r : (t + 1) * r])

    s = jnp.concatenate([m1, m2, m3], axis=0)
    out_ref[...] = jnp.concatenate([we_ref[...], s.T], axis=1)


def _tdnn_call(cit, we, ctt, w1, a2, b2, a3, b3, c3):
    n = cit.shape[1]
    r = 512
    return pl.pallas_call(
        _tdnn_body,
        grid=(n // r,),
        in_specs=[
            pl.BlockSpec((12, r), lambda i: (0, i)),
            pl.BlockSpec((r, 128), lambda i: (i, 0)),
            pl.BlockSpec((16, 128), lambda i: (0, 0)),
            pl.BlockSpec((32, 16), lambda i: (0, 0)),
            pl.BlockSpec((32, 16), lambda i: (0, 0)),
            pl.BlockSpec((32, 16), lambda i: (0, 0)),
            pl.BlockSpec((64, 16), lambda i: (0, 0)),
            pl.BlockSpec((64, 16), lambda i: (0, 0)),
            pl.BlockSpec((64, 16), lambda i: (0, 0)),
        ],
        out_specs=pl.BlockSpec((r, 256), lambda i: (i, 0)),
        out_shape=jax.ShapeDtypeStruct((n, 256), jnp.float32),
    )(cit, we, ctt, w1, a2, b2, a3, b3, c3)


def kernel(word_input, character_input, word_table, char_table, W1, W2, W3):
    b, s = word_input.shape
    n = b * s
    widx = word_input.reshape(n).astype(jnp.int32)
    we = _sc_word_gather(widx, word_table)

    cit = character_input.reshape(n, 12).astype(jnp.int32).T
    ctt = char_table.T
    out = _tdnn_call(
        cit, we, ctt,
        W1[:, :, 0],
        W2[:, :, 0], W2[:, :, 1],
        W3[:, :, 0], W3[:, :, 1], W3[:, :, 2],
    )
    return out.reshape(b, s, 256)


# speedup vs baseline: 28.6235x; 1.2075x over previous
import functools

import jax
import jax.numpy as jnp
from jax import lax
from jax.experimental import pallas as pl
from jax.experimental.pallas import tpu as pltpu
from jax.experimental.pallas import tpu_sc as plsc

_NC = 2
_NS = 16
_NW = _NC * _NS
_CH = 128


def _sc_word_gather(widx, table):
    n = widx.shape[0]
    per = n // _NW
    nch = per // _CH
    mesh = plsc.VectorSubcoreMesh(core_axis_name="c", subcore_axis_name="s")

    @functools.partial(
        pl.kernel,
        mesh=mesh,
        out_type=jax.ShapeDtypeStruct((n, 128), jnp.float32),
        scratch_types=[
            pltpu.VMEM((per,), jnp.int32),
            pltpu.VMEM((_CH, 128), jnp.float32),
            pltpu.SemaphoreType.DMA,
        ],
    )
    def k(idx_hbm, tab_hbm, out_hbm, idx_v, rows_v, sem):
        wid = lax.axis_index("s") * _NC + lax.axis_index("c")
        base = wid * per
        pltpu.sync_copy(idx_hbm.at[pl.ds(base, per)], idx_v)

        def body(j, carry):
            off = j * _CH
            pltpu.async_copy(tab_hbm.at[idx_v.at[pl.ds(off, _CH)]], rows_v, sem).wait()
            pltpu.sync_copy(rows_v, out_hbm.at[pl.ds(base + off, _CH)])
            return carry

        lax.fori_loop(0, nch, body, 0)

    return k(widx, table)


def _tdnn_body(cit_ref, we_ref, ctt_ref, w1_ref, w2_ref, w3_ref, out_ref):
    r = cit_ref.shape[1]
    cit = cit_ref[...]
    subs = lax.broadcasted_iota(jnp.int32, (128, r), 0)
    ctt = ctt_ref[...]

    es = []
    for t in range(12):
        oh = (jnp.broadcast_to(cit[t : t + 1, :], (128, r)) == subs).astype(
            jnp.float32
        )
        es.append(jnp.dot(ctt, oh, preferred_element_type=jnp.float32))
    eT = jnp.concatenate(es, axis=1)

    y1 = jnp.maximum(
        jnp.dot(w1_ref[...], eT, preferred_element_type=jnp.float32), 0.0
    )
    m1 = y1[:, 0:r]
    for t in range(1, 12):
        m1 = jnp.maximum(m1, y1[:, t * r : (t + 1) * r])

    x2 = jnp.concatenate([eT[:, 0 : 11 * r], eT[:, r : 12 * r]], axis=0)
    y2 = jnp.maximum(
        jnp.dot(w2_ref[...], x2, preferred_element_type=jnp.float32), 0.0
    )
    m2 = y2[:, 0:r]
    for t in range(1, 11):
        m2 = jnp.maximum(m2, y2[:, t * r : (t + 1) * r])

    x3 = jnp.concatenate(
        [eT[:, 0 : 10 * r], eT[:, r : 11 * r], eT[:, 2 * r : 12 * r]], axis=0
    )
    y3 = jnp.maximum(
        jnp.dot(w3_ref[...], x3, preferred_element_type=jnp.float32), 0.0
    )
    m3 = y3[:, 0:r]
    for t in range(1, 10):
        m3 = jnp.maximum(m3, y3[:, t * r : (t + 1) * r])

    s = jnp.concatenate([m1, m2, m3], axis=0)
    out_ref[...] = jnp.concatenate([we_ref[...], s.T], axis=1)


def _tdnn_call(cit, we, ctt, w1, w2, w3):
    n = cit.shape[1]
    r = 512
    return pl.pallas_call(
        _tdnn_body,
        grid=(n // r,),
        in_specs=[
            pl.BlockSpec((12, r), lambda i: (0, i)),
            pl.BlockSpec((r, 128), lambda i: (i, 0)),
            pl.BlockSpec((16, 128), lambda i: (0, 0)),
            pl.BlockSpec((32, 16), lambda i: (0, 0)),
            pl.BlockSpec((32, 32), lambda i: (0, 0)),
            pl.BlockSpec((64, 48), lambda i: (0, 0)),
        ],
        out_specs=pl.BlockSpec((r, 256), lambda i: (i, 0)),
        out_shape=jax.ShapeDtypeStruct((n, 256), jnp.float32),
    )(cit, we, ctt, w1, w2, w3)


def kernel(word_input, character_input, word_table, char_table, W1, W2, W3):
    b, s = word_input.shape
    n = b * s
    widx = word_input.reshape(n).astype(jnp.int32)
    we = _sc_word_gather(widx, word_table)

    cit = character_input.reshape(n, 12).astype(jnp.int32).T
    ctt = char_table.T
    w2 = jnp.concatenate([W2[:, :, 0], W2[:, :, 1]], axis=1)
    w3 = jnp.concatenate([W3[:, :, 0], W3[:, :, 1], W3[:, :, 2]], axis=1)
    out = _tdnn_call(cit, we, ctt, W1[:, :, 0], w2, w3)
    return out.reshape(b, s, 256)


# speedup vs baseline: 35.4109x; 1.2371x over previous
import functools

import jax
import jax.numpy as jnp
from jax import lax
from jax.experimental import pallas as pl
from jax.experimental.pallas import tpu as pltpu
from jax.experimental.pallas import tpu_sc as plsc

_NC = 2
_NS = 16
_NW = _NC * _NS
_CH = 128


def _sc_word_gather(widx, table):
    n = widx.shape[0]
    per = n // _NW
    nch = per // _CH
    mesh = plsc.VectorSubcoreMesh(core_axis_name="c", subcore_axis_name="s")

    @functools.partial(
        pl.kernel,
        mesh=mesh,
        out_type=jax.ShapeDtypeStruct((n, 128), jnp.float32),
        scratch_types=[
            pltpu.VMEM((per,), jnp.int32),
            pltpu.VMEM((_CH, 128), jnp.float32),
            pltpu.SemaphoreType.DMA,
        ],
    )
    def k(idx_hbm, tab_hbm, out_hbm, idx_v, rows_v, sem):
        wid = lax.axis_index("s") * _NC + lax.axis_index("c")
        base = wid * per
        pltpu.sync_copy(idx_hbm.at[pl.ds(base, per)], idx_v)

        def body(j, carry):
            off = j * _CH
            pltpu.async_copy(tab_hbm.at[idx_v.at[pl.ds(off, _CH)]], rows_v, sem).wait()
            pltpu.sync_copy(rows_v, out_hbm.at[pl.ds(base + off, _CH)])
            return carry

        lax.fori_loop(0, nch, body, 0)

    return k(widx, table)


def _tdnn_body(cit_ref, we_ref, ctt_ref, w1_ref, w2_ref, w3_ref, out_ref):
    r = cit_ref.shape[1]
    cit = cit_ref[...]
    subs = lax.broadcasted_iota(jnp.int32, (128, r), 0)
    ctt = ctt_ref[...]

    es = []
    for t in range(12):
        oh = (jnp.broadcast_to(cit[t : t + 1, :], (128, r)) == subs).astype(
            jnp.float32
        )
        es.append(jnp.dot(ctt, oh, preferred_element_type=jnp.float32))
    eT = jnp.concatenate(es, axis=1)

    y1 = jnp.maximum(
        jnp.dot(w1_ref[...], eT, preferred_element_type=jnp.float32), 0.0
    )
    m1 = y1[:, 0:r]
    for t in range(1, 12):
        m1 = jnp.maximum(m1, y1[:, t * r : (t + 1) * r])

    x2 = jnp.concatenate([eT[:, 0 : 11 * r], eT[:, r : 12 * r]], axis=0)
    y2 = jnp.maximum(
        jnp.dot(w2_ref[...], x2, preferred_element_type=jnp.float32), 0.0
    )
    m2 = y2[:, 0:r]
    for t in range(1, 11):
        m2 = jnp.maximum(m2, y2[:, t * r : (t + 1) * r])

    x3 = jnp.concatenate(
        [eT[:, 0 : 10 * r], eT[:, r : 11 * r], eT[:, 2 * r : 12 * r]], axis=0
    )
    y3 = jnp.maximum(
        jnp.dot(w3_ref[...], x3, preferred_element_type=jnp.float32), 0.0
    )
    m3 = y3[:, 0:r]
    for t in range(1, 10):
        m3 = jnp.maximum(m3, y3[:, t * r : (t + 1) * r])

    s = jnp.concatenate([m1, m2, m3], axis=0)
    out_ref[...] = jnp.concatenate([we_ref[...], s.T], axis=1)


def _tdnn_call(cit, we, ctt, w1, w2, w3):
    n = cit.shape[1]
    r = 1024
    return pl.pallas_call(
        _tdnn_body,
        grid=(n // r,),
        in_specs=[
            pl.BlockSpec((12, r), lambda i: (0, i)),
            pl.BlockSpec((r, 128), lambda i: (i, 0)),
            pl.BlockSpec((16, 128), lambda i: (0, 0)),
            pl.BlockSpec((32, 16), lambda i: (0, 0)),
            pl.BlockSpec((32, 32), lambda i: (0, 0)),
            pl.BlockSpec((64, 48), lambda i: (0, 0)),
        ],
        out_specs=pl.BlockSpec((r, 256), lambda i: (i, 0)),
        out_shape=jax.ShapeDtypeStruct((n, 256), jnp.float32),
    )(cit, we, ctt, w1, w2, w3)


def kernel(word_input, character_input, word_table, char_table, W1, W2, W3):
    b, s = word_input.shape
    n = b * s
    widx = word_input.reshape(n).astype(jnp.int32)
    we = _sc_word_gather(widx, word_table)

    cit = character_input.reshape(n, 12).astype(jnp.int32).T
    ctt = char_table.T
    w2 = jnp.concatenate([W2[:, :, 0], W2[:, :, 1]], axis=1)
    w3 = jnp.concatenate([W3[:, :, 0], W3[:, :, 1], W3[:, :, 2]], axis=1)
    out = _tdnn_call(cit, we, ctt, W1[:, :, 0], w2, w3)
    return out.reshape(b, s, 256)


# speedup vs baseline: 37.6064x; 1.0620x over previous
import functools

import jax
import jax.numpy as jnp
from jax import lax
from jax.experimental import pallas as pl
from jax.experimental.pallas import tpu as pltpu
from jax.experimental.pallas import tpu_sc as plsc

_NC = 2
_NS = 16
_NW = _NC * _NS
_CH = 128


def _sc_word_gather(widx, table):
    n = widx.shape[0]
    per = n // _NW
    nch = per // _CH
    mesh = plsc.VectorSubcoreMesh(core_axis_name="c", subcore_axis_name="s")

    @functools.partial(
        pl.kernel,
        mesh=mesh,
        out_type=jax.ShapeDtypeStruct((n, 128), jnp.float32),
        scratch_types=[
            pltpu.VMEM((per,), jnp.int32),
            pltpu.VMEM((_CH, 128), jnp.float32),
            pltpu.SemaphoreType.DMA,
        ],
    )
    def k(idx_hbm, tab_hbm, out_hbm, idx_v, rows_v, sem):
        wid = lax.axis_index("s") * _NC + lax.axis_index("c")
        base = wid * per
        pltpu.sync_copy(idx_hbm.at[pl.ds(base, per)], idx_v)

        def body(j, carry):
            off = j * _CH
            pltpu.async_copy(tab_hbm.at[idx_v.at[pl.ds(off, _CH)]], rows_v, sem).wait()
            pltpu.sync_copy(rows_v, out_hbm.at[pl.ds(base + off, _CH)])
            return carry

        lax.fori_loop(0, nch, body, 0)

    return k(widx, table)


def _tdnn_body(cit_ref, we_ref, ctt_ref, w1_ref, w2_ref, w3_ref, out_ref):
    r = cit_ref.shape[1]
    cit = cit_ref[...]
    subs = lax.broadcasted_iota(jnp.int32, (128, r), 0)
    ctt = ctt_ref[...]

    es = []
    for t in range(12):
        oh = (jnp.broadcast_to(cit[t : t + 1, :], (128, r)) == subs).astype(
            jnp.float32
        )
        es.append(jnp.dot(ctt, oh, preferred_element_type=jnp.float32))
    eT = jnp.concatenate(es, axis=1)

    y1 = jnp.maximum(
        jnp.dot(w1_ref[...], eT, preferred_element_type=jnp.float32), 0.0
    )
    m1 = y1[:, 0:r]
    for t in range(1, 12):
        m1 = jnp.maximum(m1, y1[:, t * r : (t + 1) * r])

    x2 = jnp.concatenate([eT[:, 0 : 11 * r], eT[:, r : 12 * r]], axis=0)
    y2 = jnp.maximum(
        jnp.dot(w2_ref[...], x2, preferred_element_type=jnp.float32), 0.0
    )
    m2 = y2[:, 0:r]
    for t in range(1, 11):
        m2 = jnp.maximum(m2, y2[:, t * r : (t + 1) * r])

    x3 = jnp.concatenate(
        [eT[:, 0 : 10 * r], eT[:, r : 11 * r], eT[:, 2 * r : 12 * r]], axis=0
    )
    y3 = jnp.maximum(
        jnp.dot(w3_ref[...], x3, preferred_element_type=jnp.float32), 0.0
    )
    m3 = y3[:, 0:r]
    for t in range(1, 10):
        m3 = jnp.maximum(m3, y3[:, t * r : (t + 1) * r])

    s = jnp.concatenate([m1, m2, m3], axis=0)
    out_ref[...] = jnp.concatenate([we_ref[...], s.T], axis=1)


def _tdnn_call(cit, we, ctt, w1, w2, w3):
    n = cit.shape[1]
    r = 2048
    return pl.pallas_call(
        _tdnn_body,
        grid=(n // r,),
        in_specs=[
            pl.BlockSpec((12, r), lambda i: (0, i)),
            pl.BlockSpec((r, 128), lambda i: (i, 0)),
            pl.BlockSpec((16, 128), lambda i: (0, 0)),
            pl.BlockSpec((32, 16), lambda i: (0, 0)),
            pl.BlockSpec((32, 32), lambda i: (0, 0)),
            pl.BlockSpec((64, 48), lambda i: (0, 0)),
        ],
        out_specs=pl.BlockSpec((r, 256), lambda i: (i, 0)),
        out_shape=jax.ShapeDtypeStruct((n, 256), jnp.float32),
    )(cit, we, ctt, w1, w2, w3)


def kernel(word_input, character_input, word_table, char_table, W1, W2, W3):
    b, s = word_input.shape
    n = b * s
    widx = word_input.reshape(n).astype(jnp.int32)
    we = _sc_word_gather(widx, word_table)

    cit = character_input.reshape(n, 12).astype(jnp.int32).T
    ctt = char_table.T
    w2 = jnp.concatenate([W2[:, :, 0], W2[:, :, 1]], axis=1)
    w3 = jnp.concatenate([W3[:, :, 0], W3[:, :, 1], W3[:, :, 2]], axis=1)
    out = _tdnn_call(cit, we, ctt, W1[:, :, 0], w2, w3)
    return out.reshape(b, s, 256)


# speedup vs baseline: 40.8718x; 1.0868x over previous
import functools

import jax
import jax.numpy as jnp
from jax import lax
from jax.experimental import pallas as pl
from jax.experimental.pallas import tpu as pltpu
from jax.experimental.pallas import tpu_sc as plsc

_NC = 2
_NS = 16
_NW = _NC * _NS
_CH = 128


def _sc_word_gather(widx, table):
    n = widx.shape[0]
    per = n // _NW
    nch = per // _CH
    mesh = plsc.VectorSubcoreMesh(core_axis_name="c", subcore_axis_name="s")

    @functools.partial(
        pl.kernel,
        mesh=mesh,
        out_type=jax.ShapeDtypeStruct((n, 128), jnp.float32),
        scratch_types=[
            pltpu.VMEM((per,), jnp.int32),
            pltpu.VMEM((2, _CH, 128), jnp.float32),
            pltpu.SemaphoreType.DMA,
            pltpu.SemaphoreType.DMA,
        ],
    )
    def k(idx_hbm, tab_hbm, out_hbm, idx_v, rows_v, g0, g1):
        wid = lax.axis_index("s") * _NC + lax.axis_index("c")
        base = wid * per
        pltpu.sync_copy(idx_hbm.at[pl.ds(base, per)], idx_v)
        gs = (g0, g1)

        def gather(j, slot):
            pltpu.async_copy(
                tab_hbm.at[idx_v.at[pl.ds(j * _CH, _CH)]], rows_v.at[slot], gs[slot]
            )

        def gwait(j, slot):
            pltpu.make_async_copy(
                tab_hbm.at[idx_v.at[pl.ds(j * _CH, _CH)]], rows_v.at[slot], gs[slot]
            ).wait()

        def store(j, slot):
            pltpu.sync_copy(rows_v.at[slot], out_hbm.at[pl.ds(base + j * _CH, _CH)])

        gather(0, 0)
        gather(1, 1)

        def body(p, carry):
            j = 2 * p
            gwait(j, 0)
            store(j, 0)

            @pl.when(j + 2 < nch)
            def _():
                gather(j + 2, 0)

            gwait(j + 1, 1)
            store(j + 1, 1)

            @pl.when(j + 3 < nch)
            def _():
                gather(j + 3, 1)

            return carry

        lax.fori_loop(0, nch // 2, body, 0)

    return k(widx, table)


def _tdnn_body(cit_ref, we_ref, ctt_ref, w1_ref, w2_ref, w3_ref, out_ref):
    r = cit_ref.shape[1]
    cit = cit_ref[...]
    subs = lax.broadcasted_iota(jnp.int32, (128, r), 0)
    ctt = ctt_ref[...]

    es = []
    for t in range(12):
        oh = (jnp.broadcast_to(cit[t : t + 1, :], (128, r)) == subs).astype(
            jnp.float32
        )
        es.append(jnp.dot(ctt, oh, preferred_element_type=jnp.float32))
    eT = jnp.concatenate(es, axis=1)

    y1 = jnp.maximum(
        jnp.dot(w1_ref[...], eT, preferred_element_type=jnp.float32), 0.0
    )
    m1 = y1[:, 0:r]
    for t in range(1, 12):
        m1 = jnp.maximum(m1, y1[:, t * r : (t + 1) * r])

    x2 = jnp.concatenate([eT[:, 0 : 11 * r], eT[:, r : 12 * r]], axis=0)
    y2 = jnp.maximum(
        jnp.dot(w2_ref[...], x2, preferred_element_type=jnp.float32), 0.0
    )
    m2 = y2[:, 0:r]
    for t in range(1, 11):
        m2 = jnp.maximum(m2, y2[:, t * r : (t + 1) * r])

    x3 = jnp.concatenate(
        [eT[:, 0 : 10 * r], eT[:, r : 11 * r], eT[:, 2 * r : 12 * r]], axis=0
    )
    y3 = jnp.maximum(
        jnp.dot(w3_ref[...], x3, preferred_element_type=jnp.float32), 0.0
    )
    m3 = y3[:, 0:r]
    for t in range(1, 10):
        m3 = jnp.maximum(m3, y3[:, t * r : (t + 1) * r])

    s = jnp.concatenate([m1, m2, m3], axis=0)
    out_ref[...] = jnp.concatenate([we_ref[...], s.T], axis=1)


def _tdnn_call(cit, we, ctt, w1, w2, w3):
    n = cit.shape[1]
    r = 2048
    return pl.pallas_call(
        _tdnn_body,
        grid=(n // r,),
        in_specs=[
            pl.BlockSpec((12, r), lambda i: (0, i)),
            pl.BlockSpec((r, 128), lambda i: (i, 0)),
            pl.BlockSpec((16, 128), lambda i: (0, 0)),
            pl.BlockSpec((32, 16), lambda i: (0, 0)),
            pl.BlockSpec((32, 32), lambda i: (0, 0)),
            pl.BlockSpec((64, 48), lambda i: (0, 0)),
        ],
        out_specs=pl.BlockSpec((r, 256), lambda i: (i, 0)),
        out_shape=jax.ShapeDtypeStruct((n, 256), jnp.float32),
    )(cit, we, ctt, w1, w2, w3)


def kernel(word_input, character_input, word_table, char_table, W1, W2, W3):
    b, s = word_input.shape
    n = b * s
    widx = word_input.reshape(n).astype(jnp.int32)
    we = _sc_word_gather(widx, word_table)

    cit = character_input.reshape(n, 12).astype(jnp.int32).T
    ctt = char_table.T
    w2 = jnp.concatenate([W2[:, :, 0], W2[:, :, 1]], axis=1)
    w3 = jnp.concatenate([W3[:, :, 0], W3[:, :, 1], W3[:, :, 2]], axis=1)
    out = _tdnn_call(cit, we, ctt, W1[:, :, 0], w2, w3)
    return out.reshape(b, s, 256)


# speedup vs baseline: 48.1708x; 1.1786x over previous
import functools

import jax
import jax.numpy as jnp
from jax import lax
from jax.experimental import pallas as pl
from jax.experimental.pallas import tpu as pltpu
from jax.experimental.pallas import tpu_sc as plsc

_NC = 2
_NS = 16
_NW = _NC * _NS
_CH = 128


def _sc_word_gather(widx, table):
    n = widx.shape[0]
    per = n // _NW
    nch = per // _CH
    mesh = plsc.VectorSubcoreMesh(core_axis_name="c", subcore_axis_name="s")

    @functools.partial(
        pl.kernel,
        mesh=mesh,
        out_type=jax.ShapeDtypeStruct((n, 128), jnp.float32),
        scratch_types=[
            pltpu.VMEM((per,), jnp.int32),
            pltpu.VMEM((2, _CH, 128), jnp.float32),
            pltpu.SemaphoreType.DMA,
            pltpu.SemaphoreType.DMA,
        ],
    )
    def k(idx_hbm, tab_hbm, out_hbm, idx_v, rows_v, g0, g1):
        wid = lax.axis_index("s") * _NC + lax.axis_index("c")
        base = wid * per
        pltpu.sync_copy(idx_hbm.at[pl.ds(base, per)], idx_v)
        gs = (g0, g1)

        def gather(j, slot):
            pltpu.async_copy(
                tab_hbm.at[idx_v.at[pl.ds(j * _CH, _CH)]], rows_v.at[slot], gs[slot]
            )

        def gwait(j, slot):
            pltpu.make_async_copy(
                tab_hbm.at[idx_v.at[pl.ds(j * _CH, _CH)]], rows_v.at[slot], gs[slot]
            ).wait()

        def store(j, slot):
            pltpu.sync_copy(rows_v.at[slot], out_hbm.at[pl.ds(base + j * _CH, _CH)])

        gather(0, 0)
        gather(1, 1)

        def body(p, carry):
            j = 2 * p
            gwait(j, 0)
            store(j, 0)

            @pl.when(j + 2 < nch)
            def _():
                gather(j + 2, 0)

            gwait(j + 1, 1)
            store(j + 1, 1)

            @pl.when(j + 3 < nch)
            def _():
                gather(j + 3, 1)

            return carry

        lax.fori_loop(0, nch // 2, body, 0)

    return k(widx, table)


def _tdnn_body(cit_ref, we_ref, ctt_ref, w1_ref, w2_ref, w3_ref, out_ref):
    r = cit_ref.shape[1]
    cit = cit_ref[...]
    subs = lax.broadcasted_iota(jnp.int32, (128, r), 0)
    ctt = ctt_ref[...]

    es = []
    for t in range(12):
        idx_t = jnp.broadcast_to(cit[t : t + 1, :], (16, r))
        es.append(jnp.take_along_axis(ctt, idx_t, axis=1))
    eT = jnp.concatenate(es, axis=1)

    y1 = jnp.maximum(
        jnp.dot(w1_ref[...], eT, preferred_element_type=jnp.float32), 0.0
    )
    m1 = y1[:, 0:r]
    for t in range(1, 12):
        m1 = jnp.maximum(m1, y1[:, t * r : (t + 1) * r])

    x2 = jnp.concatenate([eT[:, 0 : 11 * r], eT[:, r : 12 * r]], axis=0)
    y2 = jnp.maximum(
        jnp.dot(w2_ref[...], x2, preferred_element_type=jnp.float32), 0.0
    )
    m2 = y2[:, 0:r]
    for t in range(1, 11):
        m2 = jnp.maximum(m2, y2[:, t * r : (t + 1) * r])

    x3 = jnp.concatenate(
        [eT[:, 0 : 10 * r], eT[:, r : 11 * r], eT[:, 2 * r : 12 * r]], axis=0
    )
    y3 = jnp.maximum(
        jnp.dot(w3_ref[...], x3, preferred_element_type=jnp.float32), 0.0
    )
    m3 = y3[:, 0:r]
    for t in range(1, 10):
        m3 = jnp.maximum(m3, y3[:, t * r : (t + 1) * r])

    s = jnp.concatenate([m1, m2, m3], axis=0)
    out_ref[...] = jnp.concatenate([we_ref[...], s.T], axis=1)


def _tdnn_call(cit, we, ctt, w1, w2, w3):
    n = cit.shape[1]
    r = 2048
    return pl.pallas_call(
        _tdnn_body,
        grid=(n // r,),
        in_specs=[
            pl.BlockSpec((12, r), lambda i: (0, i)),
            pl.BlockSpec((r, 128), lambda i: (i, 0)),
            pl.BlockSpec((16, 128), lambda i: (0, 0)),
            pl.BlockSpec((32, 16), lambda i: (0, 0)),
            pl.BlockSpec((32, 32), lambda i: (0, 0)),
            pl.BlockSpec((64, 48), lambda i: (0, 0)),
        ],
        out_specs=pl.BlockSpec((r, 256), lambda i: (i, 0)),
        out_shape=jax.ShapeDtypeStruct((n, 256), jnp.float32),
    )(cit, we, ctt, w1, w2, w3)


def kernel(word_input, character_input, word_table, char_table, W1, W2, W3):
    b, s = word_input.shape
    n = b * s
    widx = word_input.reshape(n).astype(jnp.int32)
    we = _sc_word_gather(widx, word_table)

    cit = character_input.reshape(n, 12).astype(jnp.int32).T
    ctt = char_table.T
    w2 = jnp.concatenate([W2[:, :, 0], W2[:, :, 1]], axis=1)
    w3 = jnp.concatenate([W3[:, :, 0], W3[:, :, 1], W3[:, :, 2]], axis=1)
    out = _tdnn_call(cit, we, ctt, W1[:, :, 0], w2, w3)
    return out.reshape(b, s, 256)


# speedup vs baseline: 48.7396x; 1.0118x over previous
import functools

import jax
import jax.numpy as jnp
from jax import lax
from jax.experimental import pallas as pl
from jax.experimental.pallas import tpu as pltpu
from jax.experimental.pallas import tpu_sc as plsc

_NC = 2
_NS = 16
_NW = _NC * _NS
_CH = 128


def _sc_word_gather(widx, table):
    n = widx.shape[0]
    per = n // _NW
    nch = per // _CH
    mesh = plsc.VectorSubcoreMesh(core_axis_name="c", subcore_axis_name="s")

    @functools.partial(
        pl.kernel,
        mesh=mesh,
        out_type=jax.ShapeDtypeStruct((n, 256), jnp.float32),
        scratch_types=[
            pltpu.VMEM((per,), jnp.int32),
            pltpu.VMEM((2, _CH, 128), jnp.float32),
            pltpu.SemaphoreType.DMA,
            pltpu.SemaphoreType.DMA,
        ],
    )
    def k(idx_hbm, tab_hbm, out_hbm, idx_v, rows_v, g0, g1):
        wid = lax.axis_index("s") * _NC + lax.axis_index("c")
        base = wid * per
        pltpu.sync_copy(idx_hbm.at[pl.ds(base, per)], idx_v)
        gs = (g0, g1)

        def gather(j, slot):
            pltpu.async_copy(
                tab_hbm.at[idx_v.at[pl.ds(j * _CH, _CH)]], rows_v.at[slot], gs[slot]
            )

        def gwait(j, slot):
            pltpu.make_async_copy(
                tab_hbm.at[idx_v.at[pl.ds(j * _CH, _CH)]], rows_v.at[slot], gs[slot]
            ).wait()

        def store(j, slot):
            pltpu.sync_copy(
                rows_v.at[slot],
                out_hbm.at[pl.ds(base + j * _CH, _CH), pl.ds(0, 128)],
            )

        gather(0, 0)
        gather(1, 1)

        def body(p, carry):
            j = 2 * p
            gwait(j, 0)
            store(j, 0)

            @pl.when(j + 2 < nch)
            def _():
                gather(j + 2, 0)

            gwait(j + 1, 1)
            store(j + 1, 1)

            @pl.when(j + 3 < nch)
            def _():
                gather(j + 3, 1)

            return carry

        lax.fori_loop(0, nch // 2, body, 0)

    return k(widx, table)


def _tdnn_body(cit_ref, wide_ref, ctt_ref, w1_ref, w2_ref, w3_ref, out_ref):
    del wide_ref
    r = cit_ref.shape[1]
    cit = cit_ref[...]
    subs = lax.broadcasted_iota(jnp.int32, (128, r), 0)
    ctt = ctt_ref[...]

    es = []
    for t in range(12):
        idx_t = jnp.broadcast_to(cit[t : t + 1, :], (16, r))
        es.append(jnp.take_along_axis(ctt, idx_t, axis=1))
    eT = jnp.concatenate(es, axis=1)

    y1 = jnp.maximum(
        jnp.dot(w1_ref[...], eT, preferred_element_type=jnp.float32), 0.0
    )
    m1 = y1[:, 0:r]
    for t in range(1, 12):
        m1 = jnp.maximum(m1, y1[:, t * r : (t + 1) * r])

    x2 = jnp.concatenate([eT[:, 0 : 11 * r], eT[:, r : 12 * r]], axis=0)
    y2 = jnp.maximum(
        jnp.dot(w2_ref[...], x2, preferred_element_type=jnp.float32), 0.0
    )
    m2 = y2[:, 0:r]
    for t in range(1, 11):
        m2 = jnp.maximum(m2, y2[:, t * r : (t + 1) * r])

    x3 = jnp.concatenate(
        [eT[:, 0 : 10 * r], eT[:, r : 11 * r], eT[:, 2 * r : 12 * r]], axis=0
    )
    y3 = jnp.maximum(
        jnp.dot(w3_ref[...], x3, preferred_element_type=jnp.float32), 0.0
    )
    m3 = y3[:, 0:r]
    for t in range(1, 10):
        m3 = jnp.maximum(m3, y3[:, t * r : (t + 1) * r])

    s = jnp.concatenate([m1, m2, m3], axis=0)
    out_ref[...] = s.T


def _tdnn_call(cit, wide, ctt, w1, w2, w3):
    n = cit.shape[1]
    r = 2048
    return pl.pallas_call(
        _tdnn_body,
        grid=(n // r,),
        in_specs=[
            pl.BlockSpec((12, r), lambda i: (0, i)),
            pl.BlockSpec(memory_space=pl.ANY),
            pl.BlockSpec((16, 128), lambda i: (0, 0)),
            pl.BlockSpec((32, 16), lambda i: (0, 0)),
            pl.BlockSpec((32, 32), lambda i: (0, 0)),
            pl.BlockSpec((64, 48), lambda i: (0, 0)),
        ],
        out_specs=pl.BlockSpec((r, 128), lambda i: (i, 1)),
        out_shape=jax.ShapeDtypeStruct((n, 256), jnp.float32),
        input_output_aliases={1: 0},
    )(cit, wide, ctt, w1, w2, w3)


def kernel(word_input, character_input, word_table, char_table, W1, W2, W3):
    b, s = word_input.shape
    n = b * s
    widx = word_input.reshape(n).astype(jnp.int32)
    wide = _sc_word_gather(widx, word_table)

    cit = character_input.reshape(n, 12).astype(jnp.int32).T
    ctt = char_table.T
    w2 = jnp.concatenate([W2[:, :, 0], W2[:, :, 1]], axis=1)
    w3 = jnp.concatenate([W3[:, :, 0], W3[:, :, 1], W3[:, :, 2]], axis=1)
    out = _tdnn_call(cit, wide, ctt, W1[:, :, 0], w2, w3)
    return out.reshape(b, s, 256)
